# baseline (speedup 1.0000x reference)
import functools
import math

import jax
import jax.numpy as jnp
from jax import lax
from jax.experimental import pallas as pl
from jax.experimental.pallas import tpu as pltpu
from jax.experimental.pallas import tpu_sc as plsc

PAD = 0
SMOOTH = 0.1


def _sc_gather_part(n_rows, vocab):
    eps = SMOOTH / (vocab - 2)
    coef = (1.0 - SMOOTH) - eps
    rows_per_w = 16
    n_workers = n_rows // rows_per_w
    mesh = plsc.VectorSubcoreMesh(core_axis_name="c", subcore_axis_name="s", num_cores=1)

    @functools.partial(
        pl.kernel,
        mesh=mesh,
        out_type=[
            jax.ShapeDtypeStruct((n_workers, 16), jnp.float32),
            jax.ShapeDtypeStruct((16,), jnp.float32),
        ],
        scratch_types=[
            pltpu.VMEM((16,), jnp.int32),
            pltpu.VMEM((16, 8, 128), jnp.float32),
            pltpu.VMEM((16,), jnp.float32),
            pltpu.VMEM((16, 16), jnp.float32),
            pltpu.SemaphoreType.DMA,
        ],
    )
    def k(lp_hbm, tgt_hbm, parts_hbm, fin_hbm, tgt_v, blk_v, part_v, all_v, sem):
        c = lax.axis_index("c")
        s = lax.axis_index("s")
        lanes = lax.iota(jnp.int32, 16)

        @pl.when((c == 0) & (s < n_workers))
        def _():
            base = s * rows_per_w
            pltpu.async_copy(
                tgt_hbm.at[pl.ds(base, rows_per_w)], tgt_v, sem
            ).wait()
            tv = tgt_v[...]
            amax = ((vocab - 128) // 128) * 128
            handles = []
            c0s = []
            for i in range(rows_per_w):
                t_i = tv[i]
                c0 = jnp.minimum((t_i // 128) * 128, amax)
                c0s.append(c0)
                r0 = base + 8 * (i // 8)
                handles.append(pltpu.async_copy(
                    lp_hbm.at[pl.ds(r0, 8), pl.ds(c0, 128)],
                    blk_v.at[i], sem,
                ))
            for h in handles:
                h.wait()
            acc = jnp.zeros((16,), jnp.float32)
            for i in range(rows_per_w):
                t_i = tv[i]
                w_live = jnp.where(t_i != PAD, 1.0, 0.0)
                wv = jnp.full((16,), w_live, jnp.float32)
                cc_v = jnp.full((16,), t_i - c0s[i], jnp.int32)
                for q in range(8):
                    chunk = blk_v[i, i % 8, pl.ds(16 * q, 16)]
                    colv = 16 * q + lanes
                    acc += jnp.where(colv == cc_v, -coef * wv * chunk, 0.0)
            part_v[...] = acc
            pltpu.async_copy(part_v, parts_hbm.at[s], sem).wait()

        plsc.subcore_barrier()

        @pl.when((c == 0) & (s == 0))
        def _():
            pltpu.async_copy(parts_hbm, all_v, sem).wait()
            tot = jnp.zeros((16,), jnp.float32)
            for j in range(n_workers):
                tot += all_v[j]
            dn = lax.GatherDimensionNumbers(
                offset_dims=(), collapsed_slice_dims=(0,),
                start_index_map=(0,),
            )
            for st in (8, 4, 2, 1):
                tot = tot + lax.gather(
                    tot, (lanes ^ st).reshape(16, 1), dn,
                    slice_sizes=(1,),
                    mode=lax.GatherScatterMode.PROMISE_IN_BOUNDS,
                )
            part_v[...] = tot
            pltpu.async_copy(part_v, fin_hbm, sem).wait()

    return k


def _tc_dense_part(n_rows, vocab, row_blk):
    eps = SMOOTH / (vocab - 2)
    c_row = (1.0 - SMOOTH) * math.log(1.0 - SMOOTH) + SMOOTH * math.log(eps)
    nsteps = n_rows // row_blk

    coef = (1.0 - SMOOTH) - eps
    edge_lo = ((vocab - 128) // 128) * 128 + 128

    def body(t_ref, lp_ref, out_ref, acc_ref):
        i = pl.program_id(0)

        @pl.when(i == 0)
        def _():
            acc_ref[...] = jnp.zeros_like(acc_ref)

        s = jnp.sum(lp_ref[...], axis=1, keepdims=True)
        z = lp_ref[:, 0:1]
        per_row = jnp.where(t_ref[...] != PAD, c_row + eps * z - eps * s, 0.0)
        total = jnp.sum(per_row)
        if edge_lo < vocab:
            tail = lp_ref[:, vocab - 128:]
            cid = (vocab - 128) + lax.broadcasted_iota(
                jnp.int32, (row_blk, 128), 1
            )
            m = (t_ref[...] >= edge_lo) & (cid == t_ref[...])
            total += jnp.sum(jnp.where(m, -coef * tail, 0.0))
        acc_ref[...] += total.reshape(1, 1)

        @pl.when(i == nsteps - 1)
        def _():
            out_ref[...] = acc_ref[...]

    return pl.pallas_call(
        body,
        grid=(nsteps,),
        in_specs=[
            pl.BlockSpec((row_blk, 1), lambda i: (i, 0)),
            pl.BlockSpec((row_blk, vocab), lambda i: (i, 0)),
        ],
        out_specs=pl.BlockSpec((1, 1), lambda i: (0, 0)),
        out_shape=jax.ShapeDtypeStruct((1, 1), jnp.float32),
        scratch_shapes=[pltpu.VMEM((1, 1), jnp.float32)],
        compiler_params=pltpu.CompilerParams(
            dimension_semantics=("arbitrary",),
        ),
    )


def kernel(log_probs, target):
    vocab = log_probs.shape[-1]
    lp2 = log_probs.reshape(-1, vocab)
    n_rows = lp2.shape[0]
    tgt = target.reshape(-1)

    row_blk = 32 if n_rows % 32 == 0 else n_rows
    a = _tc_dense_part(n_rows, vocab, row_blk)(tgt.reshape(n_rows, 1), lp2)
    _, b = _sc_gather_part(n_rows, vocab)(lp2, tgt)
    return a[0, 0] + b[0]
